# Initial kernel scaffold; baseline (speedup 1.0000x reference)
import jax
import jax.numpy as jnp
from jax.experimental import pallas as pl


def kernel(x, table):
    raise NotImplementedError("write your pallas kernel here")


# speedup vs baseline: 1.6868x; 1.6868x over previous
import jax
import jax.numpy as jnp
from jax.experimental import pallas as pl

_BL = 512


def _add_kernel(x_ref, t_ref, o_ref):
    o_ref[...] = x_ref[...] + t_ref[...]


def kernel(x, table):
    B, L, H = x.shape
    nl = L // _BL
    return pl.pallas_call(
        _add_kernel,
        grid=(nl, B),
        in_specs=[
            pl.BlockSpec((1, _BL, H), lambda l, b: (b, l, 0)),
            pl.BlockSpec((_BL, H), lambda l, b: (l, 0)),
        ],
        out_specs=pl.BlockSpec((1, _BL, H), lambda l, b: (b, l, 0)),
        out_shape=jax.ShapeDtypeStruct(x.shape, x.dtype),
    )(x, table)


# speedup vs baseline: 1.9439x; 1.1524x over previous
import jax
import jax.numpy as jnp
from jax.experimental import pallas as pl

_BL = 512


def _add_kernel(x_ref, t_ref, o_ref):
    o_ref[...] = x_ref[...] + t_ref[...]


def kernel(x, table):
    B, L, H = x.shape
    nl = L // _BL
    return pl.pallas_call(
        _add_kernel,
        grid=(nl,),
        in_specs=[
            pl.BlockSpec((B, _BL, H), lambda l: (0, l, 0)),
            pl.BlockSpec((_BL, H), lambda l: (l, 0)),
        ],
        out_specs=pl.BlockSpec((B, _BL, H), lambda l: (0, l, 0)),
        out_shape=jax.ShapeDtypeStruct(x.shape, x.dtype),
    )(x, table)
